# baseline (speedup 1.0000x reference)
import functools

import jax
import jax.numpy as jnp
from jax import lax
from jax.experimental import pallas as pl
from jax.experimental.pallas import tpu as pltpu
from jax.experimental.pallas import tpu_sc as plsc

N_NODES = 10000
N_EDGES = 160000
D = 128
EH = N_EDGES // 2

NC = 2
NS = 16
NW = NC * NS
CHUNK = 128
ROWS_PER_TILE = 624
ROWS_TAIL = N_NODES - NS * ROWS_PER_TILE

_mesh = plsc.VectorSubcoreMesh(core_axis_name="c", subcore_axis_name="s")


NCH = EH // CHUNK
CPW = (NCH + NW - 1) // NW


def _make_gather(n_edges):
    nchunks = n_edges // CHUNK

    def body(a_hbm, b_hbm, idxp_hbm, o1_hbm, idx_all, rows0, rows1, rows2,
             sem0, sem1, sem2, semw0, semw1, semw2):
        c = lax.axis_index("c")
        s = lax.axis_index("s")
        wid = s * NC + c
        n_my = (nchunks - wid + NW - 1) // NW
        rows = (rows0, rows1, rows2)
        sems = (sem0, sem1, sem2)
        semws = (semw0, semw1, semw2)
        pltpu.sync_copy(idxp_hbm.at[wid], idx_all)

        def start_a(k, b):
            pltpu.async_copy(
                a_hbm.at[idx_all.at[pl.ds(2 * k * CHUNK, CHUNK)]],
                rows[b], sems[b])

        def start_wb(k, b):
            pltpu.async_copy(
                rows[b], o1_hbm.at[pl.ds((wid + k * NW) * CHUNK, CHUNK)],
                semws[b])

        def wait_wb(k, b):
            pltpu.make_async_copy(
                rows[b], o1_hbm.at[pl.ds((wid + k * NW) * CHUNK, CHUNK)],
                semws[b]).wait()

        start_a(0, 0)
        start_a(1, 1)
        for k in range(CPW):
            b = k % 3

            @pl.when(k < n_my)
            def _(k=k, b=b):
                pltpu.make_async_copy(a_hbm.at[pl.ds(0, CHUNK)],
                                      rows[b], sems[b]).wait()
                if k + 2 < CPW:
                    @pl.when(k + 2 < n_my)
                    def _():
                        if k >= 1:
                            wait_wb(k - 1, (k - 1) % 3)
                        start_a(k + 2, (k + 2) % 3)
                pltpu.sync_copy(
                    b_hbm.at[idx_all.at[pl.ds((2 * k + 1) * CHUNK, CHUNK)]],
                    rows[b], add=True)
                start_wb(k, b)

        for k in range(CPW - 4, CPW):
            @pl.when((k == n_my - 1) | (k == n_my - 2) | (k == n_my - 3))
            def _(k=k):
                wait_wb(k, k % 3)

    return functools.partial(
        pl.kernel,
        out_type=jax.ShapeDtypeStruct((n_edges, D), jnp.float32),
        mesh=_mesh,
        scratch_types=[
            pltpu.VMEM((2 * CPW * CHUNK,), jnp.int32),
            pltpu.VMEM((CHUNK, D), jnp.float32),
            pltpu.VMEM((CHUNK, D), jnp.float32),
            pltpu.VMEM((CHUNK, D), jnp.float32),
            pltpu.SemaphoreType.DMA,
            pltpu.SemaphoreType.DMA,
            pltpu.SemaphoreType.DMA,
            pltpu.SemaphoreType.DMA,
            pltpu.SemaphoreType.DMA,
            pltpu.SemaphoreType.DMA,
        ],
    )(body)


def _permute_idx(src, dst):
    pad = NW * CPW * CHUNK - EH
    s = jnp.concatenate([src, jnp.zeros((pad,), jnp.int32)])
    d = jnp.concatenate([dst, jnp.zeros((pad,), jnp.int32)])
    s = s.reshape(CPW, NW, CHUNK).transpose(1, 0, 2)
    d = d.reshape(CPW, NW, CHUNK).transpose(1, 0, 2)
    inter = jnp.stack([s, d], axis=2)
    return inter.reshape(NW, 2 * CPW * CHUNK)


_gather_half = _make_gather(EH)


def _scatter_body(ea_hbm, eb_hbm, idxpa_hbm, idxpb_hbm, zero_hbm, out_hbm,
                  idx_all, rows0, rows1, sem0, sem1, sema0, sema1, shared):
    c = lax.axis_index("c")
    s = lax.axis_index("s")
    wid = s * NC + c
    row0 = s * ROWS_PER_TILE
    tail0 = NS * ROWS_PER_TILE
    rows = (rows0, rows1)
    sems = (sem0, sem1)
    semas = (sema0, sema1)
    pltpu.sync_copy(zero_hbm.at[pl.ds(row0, ROWS_PER_TILE)],
                    shared.at[pl.ds(row0, ROWS_PER_TILE)])

    @pl.when(s == NS - 1)
    def _():
        pltpu.sync_copy(zero_hbm.at[pl.ds(tail0, ROWS_TAIL)],
                        shared.at[pl.ds(tail0, ROWS_TAIL)])

    plsc.subcore_barrier()

    nchunks = EH // CHUNK
    n_my = (nchunks - wid + NW - 1) // NW
    for e_hbm, idxp_hbm in ((ea_hbm, idxpa_hbm), (eb_hbm, idxpb_hbm)):
        pltpu.sync_copy(idxp_hbm.at[wid], idx_all)

        def start_load(k, b, e_hbm=e_hbm):
            pltpu.async_copy(e_hbm.at[pl.ds((wid + k * NW) * CHUNK, CHUNK)],
                             rows[b], sems[b])

        def start_add(k, b):
            pltpu.async_copy(
                rows[b],
                shared.at[idx_all.at[pl.ds((2 * k + 1) * CHUNK, CHUNK)]],
                semas[b], add=True)

        def wait_add(k, b):
            pltpu.make_async_copy(
                rows[b],
                shared.at[idx_all.at[pl.ds((2 * k + 1) * CHUNK, CHUNK)]],
                semas[b]).wait()

        start_load(0, 0)
        for k in range(CPW):
            b = k % 2

            @pl.when(k < n_my)
            def _(k=k, b=b, e_hbm=e_hbm, start_load=start_load,
                  start_add=start_add, wait_add=wait_add):
                pltpu.make_async_copy(e_hbm.at[pl.ds(0, CHUNK)],
                                      rows[b], sems[b]).wait()
                if k + 1 < CPW:
                    @pl.when(k + 1 < n_my)
                    def _():
                        if k >= 1:
                            wait_add(k - 1, 1 - b)
                        start_load(k + 1, 1 - b)
                start_add(k, b)

        for k in range(CPW - 3, CPW):
            @pl.when((k == n_my - 1) | (k == n_my - 2))
            def _(k=k, wait_add=wait_add):
                wait_add(k, k % 2)

    plsc.subcore_barrier()
    pltpu.sync_copy(shared.at[pl.ds(row0, ROWS_PER_TILE)],
                    out_hbm.at[c, pl.ds(row0, ROWS_PER_TILE)])

    @pl.when(s == NS - 1)
    def _():
        pltpu.sync_copy(shared.at[pl.ds(tail0, ROWS_TAIL)],
                        out_hbm.at[c, pl.ds(tail0, ROWS_TAIL)])


_scatter_call = functools.partial(
    pl.kernel,
    out_type=jax.ShapeDtypeStruct((NC, N_NODES, D), jnp.float32),
    mesh=_mesh,
    scratch_types=[
        pltpu.VMEM((2 * CPW * CHUNK,), jnp.int32),
        pltpu.VMEM((CHUNK, D), jnp.float32),
        pltpu.VMEM((CHUNK, D), jnp.float32),
        pltpu.SemaphoreType.DMA,
        pltpu.SemaphoreType.DMA,
        pltpu.SemaphoreType.DMA,
        pltpu.SemaphoreType.DMA,
        pltpu.VMEM_SHARED((N_NODES, D), jnp.float32),
    ],
)(_scatter_body)


BE = 2000
BN = 2000


def _proj_body(hn, ws, wd, out_a, out_b):
    x = hn[...]
    out_a[...] = jnp.dot(x, ws[...], preferred_element_type=jnp.float32)
    out_b[...] = jnp.dot(x, wd[...], preferred_element_type=jnp.float32)


def _edge_body(g12, he, w1, b1, w2, b2, g, bb, out):
    he_v = he[...]
    x = g12[...] + b1[...] + jnp.dot(
        he_v, w1[...], preferred_element_type=jnp.float32)
    h = jnp.maximum(x, 0.0)
    y = jnp.dot(h, w2[...], preferred_element_type=jnp.float32) + b2[...]
    mu = jnp.mean(y, axis=-1, keepdims=True)
    yc = y - mu
    var = jnp.mean(yc * yc, axis=-1, keepdims=True)
    out[...] = he_v + yc * lax.rsqrt(var + 1e-5) * g[...] + bb[...]


def _node_body(hn, p0, p1, w1a, w1b, b1, w2, b2, g, bb, ws, wd,
               out_h, out_a, out_b):
    hn_v = hn[...]
    agg = p0[...] + p1[...]
    x = (jnp.dot(hn_v, w1a[...], preferred_element_type=jnp.float32)
         + jnp.dot(agg, w1b[...], preferred_element_type=jnp.float32)
         + b1[...])
    h = jnp.maximum(x, 0.0)
    y = jnp.dot(h, w2[...], preferred_element_type=jnp.float32) + b2[...]
    mu = jnp.mean(y, axis=-1, keepdims=True)
    yc = y - mu
    var = jnp.mean(yc * yc, axis=-1, keepdims=True)
    hn_new = hn_v + yc * lax.rsqrt(var + 1e-5) * g[...] + bb[...]
    out_h[...] = hn_new
    out_a[...] = jnp.dot(hn_new, ws[...], preferred_element_type=jnp.float32)
    out_b[...] = jnp.dot(hn_new, wd[...], preferred_element_type=jnp.float32)


def _row_spec(bs):
    return pl.BlockSpec((bs, D), lambda i: (i, 0))


def _w_spec():
    return pl.BlockSpec((D, D), lambda i: (0, 0))


def _b_spec():
    return pl.BlockSpec((1, D), lambda i: (0, 0))


_proj_call = pl.pallas_call(
    _proj_body,
    grid=(N_NODES // BN,),
    in_specs=[_row_spec(BN), _w_spec(), _w_spec()],
    out_specs=[_row_spec(BN), _row_spec(BN)],
    out_shape=[jax.ShapeDtypeStruct((N_NODES, D), jnp.float32)] * 2,
)

_edge_call = pl.pallas_call(
    _edge_body,
    grid=(EH // BE,),
    in_specs=[_row_spec(BE), _row_spec(BE),
              _w_spec(), _b_spec(), _w_spec(), _b_spec(),
              _b_spec(), _b_spec()],
    out_specs=_row_spec(BE),
    out_shape=jax.ShapeDtypeStruct((EH, D), jnp.float32),
)

_node_call = pl.pallas_call(
    _node_body,
    grid=(N_NODES // BN,),
    in_specs=[_row_spec(BN), _row_spec(BN), _row_spec(BN),
              _w_spec(), _w_spec(), _b_spec(), _w_spec(), _b_spec(),
              _b_spec(), _b_spec(), _w_spec(), _w_spec()],
    out_specs=[_row_spec(BN), _row_spec(BN), _row_spec(BN)],
    out_shape=[jax.ShapeDtypeStruct((N_NODES, D), jnp.float32)] * 3,
)


def kernel(h_node, h_edge, edge_index, We1, be1, We2, be2, ge, bbe,
           Wn1, bn1, Wn2, bn2, gn, bbn):
    src_a, src_b = edge_index[0, :EH], edge_index[0, EH:]
    dst_a, dst_b = edge_index[1, :EH], edge_index[1, EH:]
    idxp_a = _permute_idx(src_a, dst_a)
    idxp_b = _permute_idx(src_b, dst_b)
    he_a, he_b = h_edge[:EH], h_edge[EH:]
    zeros = jnp.zeros((N_NODES, D), jnp.float32)
    num_convs = We1.shape[0]

    a_proj, b_proj = _proj_call(h_node, We1[0, :D], We1[0, D:2 * D])
    for i in range(num_convs):
        ew = (We1[i, 2 * D:], be1[i].reshape(1, D), We2[i],
              be2[i].reshape(1, D), ge[i].reshape(1, D), bbe[i].reshape(1, D))
        g_a = _gather_half(a_proj, b_proj, idxp_a)
        g_b = _gather_half(a_proj, b_proj, idxp_b)
        he_a = _edge_call(g_a, he_a, *ew)
        he_b = _edge_call(g_b, he_b, *ew)
        partials = _scatter_call(he_a, he_b, idxp_a, idxp_b, zeros)
        j = min(i + 1, num_convs - 1)
        h_node, a_proj, b_proj = _node_call(
            h_node, partials[0], partials[1],
            Wn1[i, :D], Wn1[i, D:], bn1[i].reshape(1, D),
            Wn2[i], bn2[i].reshape(1, D),
            gn[i].reshape(1, D), bbn[i].reshape(1, D),
            We1[j, :D], We1[j, D:2 * D])
    return h_node, jnp.concatenate([he_a, he_b], axis=0)


# speedup vs baseline: 1.0008x; 1.0008x over previous
import functools

import jax
import jax.numpy as jnp
from jax import lax
from jax.experimental import pallas as pl
from jax.experimental.pallas import tpu as pltpu
from jax.experimental.pallas import tpu_sc as plsc

N_NODES = 10000
N_EDGES = 160000
D = 128
EH = N_EDGES // 2

NC = 2
NS = 16
NW = NC * NS
CHUNK = 128
ROWS_PER_TILE = 624
ROWS_TAIL = N_NODES - NS * ROWS_PER_TILE

_mesh = plsc.VectorSubcoreMesh(core_axis_name="c", subcore_axis_name="s")


NCH = EH // CHUNK
CPW = (NCH + NW - 1) // NW


def _make_gather(n_edges):
    nchunks = n_edges // CHUNK

    def body(a_hbm, b_hbm, idxp_hbm, o1_hbm, idx_all, rows0, rows1,
             sem0, sem1, semw0, semw1):
        c = lax.axis_index("c")
        s = lax.axis_index("s")
        wid = s * NC + c
        n_my = (nchunks - wid + NW - 1) // NW
        rows = (rows0, rows1)
        sems = (sem0, sem1)
        semws = (semw0, semw1)
        pltpu.sync_copy(idxp_hbm.at[wid], idx_all)

        def start_a(k, b):
            pltpu.async_copy(
                a_hbm.at[idx_all.at[pl.ds(2 * k * CHUNK, CHUNK)]],
                rows[b], sems[b])

        def start_wb(k, b):
            pltpu.async_copy(
                rows[b], o1_hbm.at[pl.ds((wid + k * NW) * CHUNK, CHUNK)],
                semws[b])

        def wait_wb(k, b):
            pltpu.make_async_copy(
                rows[b], o1_hbm.at[pl.ds((wid + k * NW) * CHUNK, CHUNK)],
                semws[b]).wait()

        start_a(0, 0)
        for k in range(CPW):
            b = k % 2

            @pl.when(k < n_my)
            def _(k=k, b=b):
                pltpu.make_async_copy(a_hbm.at[pl.ds(0, CHUNK)],
                                      rows[b], sems[b]).wait()
                if k + 1 < CPW:
                    @pl.when(k + 1 < n_my)
                    def _():
                        if k >= 1:
                            wait_wb(k - 1, 1 - b)
                        start_a(k + 1, 1 - b)
                pltpu.sync_copy(
                    b_hbm.at[idx_all.at[pl.ds((2 * k + 1) * CHUNK, CHUNK)]],
                    rows[b], add=True)
                start_wb(k, b)

        for k in range(CPW - 3, CPW):
            @pl.when((k == n_my - 1) | (k == n_my - 2))
            def _(k=k):
                wait_wb(k, k % 2)

    return functools.partial(
        pl.kernel,
        out_type=jax.ShapeDtypeStruct((n_edges, D), jnp.float32),
        mesh=_mesh,
        scratch_types=[
            pltpu.VMEM((2 * CPW * CHUNK,), jnp.int32),
            pltpu.VMEM((CHUNK, D), jnp.float32),
            pltpu.VMEM((CHUNK, D), jnp.float32),
            pltpu.SemaphoreType.DMA,
            pltpu.SemaphoreType.DMA,
            pltpu.SemaphoreType.DMA,
            pltpu.SemaphoreType.DMA,
        ],
    )(body)


def _permute_idx(src, dst):
    pad = NW * CPW * CHUNK - EH
    s = jnp.concatenate([src, jnp.zeros((pad,), jnp.int32)])
    d = jnp.concatenate([dst, jnp.zeros((pad,), jnp.int32)])
    s = s.reshape(CPW, NW, CHUNK).transpose(1, 0, 2)
    d = d.reshape(CPW, NW, CHUNK).transpose(1, 0, 2)
    inter = jnp.stack([s, d], axis=2)
    return inter.reshape(NW, 2 * CPW * CHUNK)


_gather_half = _make_gather(EH)


def _scatter_body(ea_hbm, eb_hbm, idxpa_hbm, idxpb_hbm, zero_hbm, out_hbm,
                  idx_all, rows0, rows1, sem0, sem1, sema0, sema1, shared):
    c = lax.axis_index("c")
    s = lax.axis_index("s")
    wid = s * NC + c
    row0 = s * ROWS_PER_TILE
    tail0 = NS * ROWS_PER_TILE
    rows = (rows0, rows1)
    sems = (sem0, sem1)
    semas = (sema0, sema1)
    pltpu.sync_copy(zero_hbm.at[pl.ds(row0, ROWS_PER_TILE)],
                    shared.at[pl.ds(row0, ROWS_PER_TILE)])

    @pl.when(s == NS - 1)
    def _():
        pltpu.sync_copy(zero_hbm.at[pl.ds(tail0, ROWS_TAIL)],
                        shared.at[pl.ds(tail0, ROWS_TAIL)])

    plsc.subcore_barrier()

    nchunks = EH // CHUNK
    n_my = (nchunks - wid + NW - 1) // NW
    for e_hbm, idxp_hbm in ((ea_hbm, idxpa_hbm), (eb_hbm, idxpb_hbm)):
        pltpu.sync_copy(idxp_hbm.at[wid], idx_all)

        def start_load(k, b, e_hbm=e_hbm):
            pltpu.async_copy(e_hbm.at[pl.ds((wid + k * NW) * CHUNK, CHUNK)],
                             rows[b], sems[b])

        def start_add(k, b):
            pltpu.async_copy(
                rows[b],
                shared.at[idx_all.at[pl.ds((2 * k + 1) * CHUNK, CHUNK)]],
                semas[b], add=True)

        def wait_add(k, b):
            pltpu.make_async_copy(
                rows[b],
                shared.at[idx_all.at[pl.ds((2 * k + 1) * CHUNK, CHUNK)]],
                semas[b]).wait()

        start_load(0, 0)
        for k in range(CPW):
            b = k % 2

            @pl.when(k < n_my)
            def _(k=k, b=b, e_hbm=e_hbm, start_load=start_load,
                  start_add=start_add, wait_add=wait_add):
                pltpu.make_async_copy(e_hbm.at[pl.ds(0, CHUNK)],
                                      rows[b], sems[b]).wait()
                if k + 1 < CPW:
                    @pl.when(k + 1 < n_my)
                    def _():
                        if k >= 1:
                            wait_add(k - 1, 1 - b)
                        start_load(k + 1, 1 - b)
                start_add(k, b)

        for k in range(CPW - 3, CPW):
            @pl.when((k == n_my - 1) | (k == n_my - 2))
            def _(k=k, wait_add=wait_add):
                wait_add(k, k % 2)

    plsc.subcore_barrier()
    pltpu.sync_copy(shared.at[pl.ds(row0, ROWS_PER_TILE)],
                    out_hbm.at[c, pl.ds(row0, ROWS_PER_TILE)])

    @pl.when(s == NS - 1)
    def _():
        pltpu.sync_copy(shared.at[pl.ds(tail0, ROWS_TAIL)],
                        out_hbm.at[c, pl.ds(tail0, ROWS_TAIL)])


_scatter_call = functools.partial(
    pl.kernel,
    out_type=jax.ShapeDtypeStruct((NC, N_NODES, D), jnp.float32),
    mesh=_mesh,
    scratch_types=[
        pltpu.VMEM((2 * CPW * CHUNK,), jnp.int32),
        pltpu.VMEM((CHUNK, D), jnp.float32),
        pltpu.VMEM((CHUNK, D), jnp.float32),
        pltpu.SemaphoreType.DMA,
        pltpu.SemaphoreType.DMA,
        pltpu.SemaphoreType.DMA,
        pltpu.SemaphoreType.DMA,
        pltpu.VMEM_SHARED((N_NODES, D), jnp.float32),
    ],
)(_scatter_body)


BE = 2000
BN = 2000


def _proj_body(hn, ws, wd, out_a, out_b):
    x = hn[...]
    out_a[...] = jnp.dot(x, ws[...], preferred_element_type=jnp.float32)
    out_b[...] = jnp.dot(x, wd[...], preferred_element_type=jnp.float32)


def _edge_body(g12, he, w1, b1, w2, b2, g, bb, out):
    he_v = he[...]
    x = g12[...] + b1[...] + jnp.dot(
        he_v, w1[...], preferred_element_type=jnp.float32)
    h = jnp.maximum(x, 0.0)
    y = jnp.dot(h, w2[...], preferred_element_type=jnp.float32) + b2[...]
    mu = jnp.mean(y, axis=-1, keepdims=True)
    yc = y - mu
    var = jnp.mean(yc * yc, axis=-1, keepdims=True)
    out[...] = he_v + yc * lax.rsqrt(var + 1e-5) * g[...] + bb[...]


def _node_body(hn, p0, p1, w1a, w1b, b1, w2, b2, g, bb, ws, wd,
               out_h, out_a, out_b):
    hn_v = hn[...]
    agg = p0[...] + p1[...]
    x = (jnp.dot(hn_v, w1a[...], preferred_element_type=jnp.float32)
         + jnp.dot(agg, w1b[...], preferred_element_type=jnp.float32)
         + b1[...])
    h = jnp.maximum(x, 0.0)
    y = jnp.dot(h, w2[...], preferred_element_type=jnp.float32) + b2[...]
    mu = jnp.mean(y, axis=-1, keepdims=True)
    yc = y - mu
    var = jnp.mean(yc * yc, axis=-1, keepdims=True)
    hn_new = hn_v + yc * lax.rsqrt(var + 1e-5) * g[...] + bb[...]
    out_h[...] = hn_new
    out_a[...] = jnp.dot(hn_new, ws[...], preferred_element_type=jnp.float32)
    out_b[...] = jnp.dot(hn_new, wd[...], preferred_element_type=jnp.float32)


def _row_spec(bs):
    return pl.BlockSpec((bs, D), lambda i: (i, 0))


def _w_spec():
    return pl.BlockSpec((D, D), lambda i: (0, 0))


def _b_spec():
    return pl.BlockSpec((1, D), lambda i: (0, 0))


_proj_call = pl.pallas_call(
    _proj_body,
    grid=(N_NODES // BN,),
    in_specs=[_row_spec(BN), _w_spec(), _w_spec()],
    out_specs=[_row_spec(BN), _row_spec(BN)],
    out_shape=[jax.ShapeDtypeStruct((N_NODES, D), jnp.float32)] * 2,
)

_edge_call = pl.pallas_call(
    _edge_body,
    grid=(EH // BE,),
    in_specs=[_row_spec(BE), _row_spec(BE),
              _w_spec(), _b_spec(), _w_spec(), _b_spec(),
              _b_spec(), _b_spec()],
    out_specs=_row_spec(BE),
    out_shape=jax.ShapeDtypeStruct((EH, D), jnp.float32),
)

_node_call = pl.pallas_call(
    _node_body,
    grid=(N_NODES // BN,),
    in_specs=[_row_spec(BN), _row_spec(BN), _row_spec(BN),
              _w_spec(), _w_spec(), _b_spec(), _w_spec(), _b_spec(),
              _b_spec(), _b_spec(), _w_spec(), _w_spec()],
    out_specs=[_row_spec(BN), _row_spec(BN), _row_spec(BN)],
    out_shape=[jax.ShapeDtypeStruct((N_NODES, D), jnp.float32)] * 3,
)


def kernel(h_node, h_edge, edge_index, We1, be1, We2, be2, ge, bbe,
           Wn1, bn1, Wn2, bn2, gn, bbn):
    src_a, src_b = edge_index[0, :EH], edge_index[0, EH:]
    dst_a, dst_b = edge_index[1, :EH], edge_index[1, EH:]
    idxp_a = _permute_idx(src_a, dst_a)
    idxp_b = _permute_idx(src_b, dst_b)
    he_a, he_b = h_edge[:EH], h_edge[EH:]
    zeros = jnp.zeros((N_NODES, D), jnp.float32)
    num_convs = We1.shape[0]

    a_proj, b_proj = _proj_call(h_node, We1[0, :D], We1[0, D:2 * D])
    for i in range(num_convs):
        ew = (We1[i, 2 * D:], be1[i].reshape(1, D), We2[i],
              be2[i].reshape(1, D), ge[i].reshape(1, D), bbe[i].reshape(1, D))
        g_a = _gather_half(a_proj, b_proj, idxp_a)
        g_b = _gather_half(a_proj, b_proj, idxp_b)
        he_a = _edge_call(g_a, he_a, *ew)
        he_b = _edge_call(g_b, he_b, *ew)
        partials = _scatter_call(he_a, he_b, idxp_a, idxp_b, zeros)
        j = min(i + 1, num_convs - 1)
        h_node, a_proj, b_proj = _node_call(
            h_node, partials[0], partials[1],
            Wn1[i, :D], Wn1[i, D:], bn1[i].reshape(1, D),
            Wn2[i], bn2[i].reshape(1, D),
            gn[i].reshape(1, D), bbn[i].reshape(1, D),
            We1[j, :D], We1[j, D:2 * D])
    return h_node, jnp.concatenate([he_a, he_b], axis=0)
